# initial kernel scaffold (unmeasured)
import jax
import jax.numpy as jnp
from jax import lax
from jax.experimental import pallas as pl
from jax.experimental.pallas import tpu as pltpu

N_DEV = 4
B_LOC = 2
SQ = 512
SKV = 512
H_LOC = 8
DH = 64
D_MODEL = 768
D_BLOCK = H_LOC * DH

_CompilerParams = getattr(pltpu, "CompilerParams", None) or getattr(
    pltpu, "TPUCompilerParams"
)


def _body(x_ref, wq_ref, k_ref, v_ref, wo_ref, out_ref,
          wq_buf, wo_buf, q_scr,
          wq_send, wq_recv, wo_send, wo_recv):
    my = lax.axis_index("i")
    left = (my - 1) % N_DEV
    right = (my + 1) % N_DEV

    bar = pltpu.get_barrier_semaphore()
    pl.semaphore_signal(bar, inc=1, device_id=(left,),
                        device_id_type=pl.DeviceIdType.MESH)
    pl.semaphore_signal(bar, inc=1, device_id=(right,),
                        device_id_type=pl.DeviceIdType.MESH)
    pl.semaphore_wait(bar, 2)

    qi = lax.broadcasted_iota(jnp.int32, (SQ, SKV), 0)
    ki = lax.broadcasted_iota(jnp.int32, (SQ, SKV), 1)
    mask = (jnp.abs(qi - ki) <= 128) | (ki < 32) | (qi < 32)

    for s in range(N_DEV):
        wq_src = wq_ref if s == 0 else wq_buf.at[s - 1]
        wo_src = wo_ref if s == 0 else wo_buf.at[s - 1]
        if s < N_DEV - 1:
            rd_wq = pltpu.make_async_remote_copy(
                src_ref=wq_src, dst_ref=wq_buf.at[s],
                send_sem=wq_send.at[s], recv_sem=wq_recv.at[s],
                device_id=(right,), device_id_type=pl.DeviceIdType.MESH,
            )
            rd_wq.start()
            rd_wo = pltpu.make_async_remote_copy(
                src_ref=wo_src, dst_ref=wo_buf.at[s],
                send_sem=wo_send.at[s], recv_sem=wo_recv.at[s],
                device_id=(right,), device_id_type=pl.DeviceIdType.MESH,
            )
            rd_wo.start()

        hb = (my - s) % N_DEV
        wq_s = wq_src[...]

        for b in range(B_LOC):
            qb = jnp.dot(x_ref[b], wq_s,
                         preferred_element_type=jnp.float32)
            for h in range(H_LOC):
                q_scr[h] = qb[:, h * DH:(h + 1) * DH]

            if s == 0:
                out_ref[b] = jnp.zeros((SQ, D_MODEL), jnp.float32)

            def hbody(h, carry, b=b, hb=hb, wo_src=wo_src):
                head = hb * H_LOC + h
                qh = q_scr[h]
                kh = k_ref[head, b]
                sc = lax.dot_general(
                    qh, kh, (((1,), (1,)), ((), ())),
                    preferred_element_type=jnp.float32)
                sc = jnp.where(mask, sc * 0.125, -1e9)
                m = jnp.max(sc, axis=1, keepdims=True)
                e = jnp.exp(sc - m)
                w = e / jnp.sum(e, axis=1, keepdims=True)
                vh = v_ref[head, b]
                ctx = jnp.dot(w, vh,
                              preferred_element_type=jnp.float32)
                wo_h = wo_src[pl.ds(h * DH, DH), :]
                out_ref[b] = out_ref[b] + jnp.dot(
                    ctx, wo_h, preferred_element_type=jnp.float32)
                return carry

            lax.fori_loop(0, H_LOC, hbody, 0)

        if s < N_DEV - 1:
            rd_wq.wait()
            rd_wo.wait()


def kernel(x, Wq, K_ext, V_ext, Wo):
    my = lax.axis_index("i")
    Kb = lax.dynamic_slice_in_dim(K_ext, my * B_LOC, B_LOC, axis=0)
    Vb = lax.dynamic_slice_in_dim(V_ext, my * B_LOC, B_LOC, axis=0)
    Kt = jnp.transpose(Kb, (2, 0, 1, 3))
    Vt = jnp.transpose(Vb, (2, 0, 1, 3))

    return pl.pallas_call(
        _body,
        out_shape=jax.ShapeDtypeStruct((B_LOC, SQ, D_MODEL), jnp.float32),
        in_specs=[pl.BlockSpec(memory_space=pltpu.VMEM)] * 5,
        out_specs=pl.BlockSpec(memory_space=pltpu.VMEM),
        scratch_shapes=[
            pltpu.VMEM((N_DEV - 1, D_MODEL, D_BLOCK), jnp.float32),
            pltpu.VMEM((N_DEV - 1, D_BLOCK, D_MODEL), jnp.float32),
            pltpu.VMEM((H_LOC, SQ, DH), jnp.float32),
            pltpu.SemaphoreType.DMA((N_DEV - 1,)),
            pltpu.SemaphoreType.DMA((N_DEV - 1,)),
            pltpu.SemaphoreType.DMA((N_DEV - 1,)),
            pltpu.SemaphoreType.DMA((N_DEV - 1,)),
        ],
        compiler_params=_CompilerParams(collective_id=0),
    )(x, Wq, Kt, Vt, Wo)


# baseline (device time: 172882 ns/iter reference)
import jax
import jax.numpy as jnp
from jax import lax
from jax.experimental import pallas as pl
from jax.experimental.pallas import tpu as pltpu

N_DEV = 4
B_LOC = 2
SQ = 512
SKV = 512
H_LOC = 8
DH = 64
D_MODEL = 768
D_BLOCK = H_LOC * DH

_CompilerParams = getattr(pltpu, "CompilerParams", None) or getattr(
    pltpu, "TPUCompilerParams"
)


def _body(x_ref, wq_ref, k_ref, v_ref, wo_ref, out_ref,
          wq_buf, wo_buf, q_scr,
          wq_send, wq_recv, wo_send, wo_recv):
    my = lax.axis_index("i")
    left = (my - 1) % N_DEV
    right = (my + 1) % N_DEV

    bar = pltpu.get_barrier_semaphore()
    pl.semaphore_signal(bar, inc=1, device_id=(left,),
                        device_id_type=pl.DeviceIdType.MESH)
    pl.semaphore_signal(bar, inc=1, device_id=(right,),
                        device_id_type=pl.DeviceIdType.MESH)
    pl.semaphore_wait(bar, 2)

    qi = lax.broadcasted_iota(jnp.int32, (SQ, SKV), 0)
    ki = lax.broadcasted_iota(jnp.int32, (SQ, SKV), 1)
    mask = (jnp.abs(qi - ki) <= 128) | (ki < 32) | (qi < 32)

    for s in range(N_DEV):
        wq_src = wq_ref if s == 0 else wq_buf.at[s - 1]
        wo_src = wo_ref if s == 0 else wo_buf.at[s - 1]
        if s < N_DEV - 1:
            rd_wq = pltpu.make_async_remote_copy(
                src_ref=wq_src, dst_ref=wq_buf.at[s],
                send_sem=wq_send.at[s], recv_sem=wq_recv.at[s],
                device_id=(right,), device_id_type=pl.DeviceIdType.MESH,
            )
            rd_wq.start()
            rd_wo = pltpu.make_async_remote_copy(
                src_ref=wo_src, dst_ref=wo_buf.at[s],
                send_sem=wo_send.at[s], recv_sem=wo_recv.at[s],
                device_id=(right,), device_id_type=pl.DeviceIdType.MESH,
            )
            rd_wo.start()

        hb = (my - s) % N_DEV
        wq_s = wq_src[...]

        for b in range(B_LOC):
            qb = jnp.dot(x_ref[b], wq_s,
                         preferred_element_type=jnp.float32)
            for h in range(H_LOC):
                q_scr[h] = qb[:, h * DH:(h + 1) * DH]

            if s == 0:
                out_ref[b] = jnp.zeros((SQ, D_MODEL), jnp.float32)

            def hbody(h, carry, b=b, hb=hb, wo_src=wo_src):
                head = hb * H_LOC + h
                qh = q_scr[h]
                kh = k_ref[head, b]
                sc = lax.dot_general(
                    qh, kh, (((1,), (1,)), ((), ())),
                    preferred_element_type=jnp.float32)
                sc = jnp.where(mask, sc * 0.125, -1e9)
                m = jnp.max(sc, axis=1, keepdims=True)
                e = jnp.exp(sc - m)
                w = e / jnp.sum(e, axis=1, keepdims=True)
                vh = v_ref[head, b]
                ctx = jnp.dot(w, vh,
                              preferred_element_type=jnp.float32)
                wo_h = wo_src[pl.ds(h * DH, DH), :]
                out_ref[b] = out_ref[b] + jnp.dot(
                    ctx, wo_h, preferred_element_type=jnp.float32)
                return carry

            lax.fori_loop(0, H_LOC, hbody, 0)

        if s < N_DEV - 1:
            rd_wq.wait()
            rd_wo.wait()


def kernel(x, Wq, K_ext, V_ext, Wo):
    my = lax.axis_index("i")
    Kb = lax.dynamic_slice_in_dim(K_ext, my * B_LOC, B_LOC, axis=0)
    Vb = lax.dynamic_slice_in_dim(V_ext, my * B_LOC, B_LOC, axis=0)
    Kt = jnp.transpose(Kb, (2, 0, 1, 3))
    Vt = jnp.transpose(Vb, (2, 0, 1, 3))

    return pl.pallas_call(
        _body,
        out_shape=jax.ShapeDtypeStruct((B_LOC, SQ, D_MODEL), jnp.float32),
        in_specs=[pl.BlockSpec(memory_space=pltpu.VMEM)] * 5,
        out_specs=pl.BlockSpec(memory_space=pltpu.VMEM),
        scratch_shapes=[
            pltpu.VMEM((N_DEV - 1, D_MODEL, D_BLOCK), jnp.float32),
            pltpu.VMEM((N_DEV - 1, D_BLOCK, D_MODEL), jnp.float32),
            pltpu.VMEM((H_LOC, SQ, DH), jnp.float32),
            pltpu.SemaphoreType.DMA((N_DEV - 1,)),
            pltpu.SemaphoreType.DMA((N_DEV - 1,)),
            pltpu.SemaphoreType.DMA((N_DEV - 1,)),
            pltpu.SemaphoreType.DMA((N_DEV - 1,)),
        ],
        compiler_params=_CompilerParams(
            collective_id=0, vmem_limit_bytes=100 * 1024 * 1024),
    )(x, Wq, Kt, Vt, Wo)


# device time: 87849 ns/iter; 1.9679x vs baseline; 1.9679x over previous
import jax
import jax.numpy as jnp
from jax import lax
from jax.experimental import pallas as pl
from jax.experimental.pallas import tpu as pltpu

N_DEV = 4
B_LOC = 2
SQ = 512
SKV = 512
H_LOC = 8
DH = 64
D_MODEL = 768
D_BLOCK = H_LOC * DH

_CompilerParams = getattr(pltpu, "CompilerParams", None) or getattr(
    pltpu, "TPUCompilerParams"
)


def _body(x_ref, wq_ref, k_ref, v_ref, wo_ref, out_ref,
          wq_buf, wo_buf, ctx_scr,
          wq_send, wq_recv, wo_send, wo_recv):
    my = lax.axis_index("i")
    left = (my - 1) % N_DEV
    right = (my + 1) % N_DEV

    bar = pltpu.get_barrier_semaphore()
    pl.semaphore_signal(bar, inc=1, device_id=(left,),
                        device_id_type=pl.DeviceIdType.MESH)
    pl.semaphore_signal(bar, inc=1, device_id=(right,),
                        device_id_type=pl.DeviceIdType.MESH)
    pl.semaphore_wait(bar, 2)

    qi = lax.broadcasted_iota(jnp.int32, (SQ, SKV), 0)
    ki = lax.broadcasted_iota(jnp.int32, (SQ, SKV), 1)
    mask = (jnp.abs(qi - ki) <= 128) | (ki < 32) | (qi < 32)

    for s in range(N_DEV):
        wq_src = wq_ref if s == 0 else wq_buf.at[s - 1]
        wo_src = wo_ref if s == 0 else wo_buf.at[s - 1]
        if s < N_DEV - 1:
            rd_wq = pltpu.make_async_remote_copy(
                src_ref=wq_src, dst_ref=wq_buf.at[s],
                send_sem=wq_send.at[s], recv_sem=wq_recv.at[s],
                device_id=(right,), device_id_type=pl.DeviceIdType.MESH,
            )
            rd_wq.start()
            rd_wo = pltpu.make_async_remote_copy(
                src_ref=wo_src, dst_ref=wo_buf.at[s],
                send_sem=wo_send.at[s], recv_sem=wo_recv.at[s],
                device_id=(right,), device_id_type=pl.DeviceIdType.MESH,
            )
            rd_wo.start()

        hb = (my - s) % N_DEV
        wq_s = wq_src[...]
        wo_s = wo_src[...]

        for b in range(B_LOC):
            qb = jnp.dot(x_ref[b], wq_s,
                         preferred_element_type=jnp.float32)
            for h in range(H_LOC):
                head = hb * H_LOC + h
                qh = qb[:, h * DH:(h + 1) * DH]
                kh = k_ref[head, b]
                sc = lax.dot_general(
                    qh, kh, (((1,), (1,)), ((), ())),
                    preferred_element_type=jnp.float32)
                sc = jnp.where(mask, sc * 0.125, -1e9)
                m = jnp.max(sc, axis=1, keepdims=True)
                e = jnp.exp(sc - m)
                w = e / jnp.sum(e, axis=1, keepdims=True)
                vh = v_ref[head, b]
                ctx_scr[:, h * DH:(h + 1) * DH] = jnp.dot(
                    w, vh, preferred_element_type=jnp.float32)
            contrib = jnp.dot(ctx_scr[...], wo_s,
                              preferred_element_type=jnp.float32)
            if s == 0:
                out_ref[b] = contrib
            else:
                out_ref[b] = out_ref[b] + contrib

        if s < N_DEV - 1:
            rd_wq.wait()
            rd_wo.wait()


def kernel(x, Wq, K_ext, V_ext, Wo):
    my = lax.axis_index("i")
    Kb = lax.dynamic_slice_in_dim(K_ext, my * B_LOC, B_LOC, axis=0)
    Vb = lax.dynamic_slice_in_dim(V_ext, my * B_LOC, B_LOC, axis=0)
    Kt = jnp.transpose(Kb, (2, 0, 1, 3))
    Vt = jnp.transpose(Vb, (2, 0, 1, 3))

    return pl.pallas_call(
        _body,
        out_shape=jax.ShapeDtypeStruct((B_LOC, SQ, D_MODEL), jnp.float32),
        in_specs=[pl.BlockSpec(memory_space=pltpu.VMEM)] * 5,
        out_specs=pl.BlockSpec(memory_space=pltpu.VMEM),
        scratch_shapes=[
            pltpu.VMEM((N_DEV - 1, D_MODEL, D_BLOCK), jnp.float32),
            pltpu.VMEM((N_DEV - 1, D_BLOCK, D_MODEL), jnp.float32),
            pltpu.VMEM((SQ, D_BLOCK), jnp.float32),
            pltpu.SemaphoreType.DMA((N_DEV - 1,)),
            pltpu.SemaphoreType.DMA((N_DEV - 1,)),
            pltpu.SemaphoreType.DMA((N_DEV - 1,)),
            pltpu.SemaphoreType.DMA((N_DEV - 1,)),
        ],
        compiler_params=_CompilerParams(
            collective_id=0, vmem_limit_bytes=100 * 1024 * 1024),
    )(x, Wq, Kt, Vt, Wo)
